# baseline (speedup 1.0000x reference)
import functools

import jax
import jax.numpy as jnp
from jax import lax
from jax.experimental import pallas as pl
from jax.experimental.pallas import tpu as pltpu
from jax.experimental.pallas import tpu_sc as plsc

BATCH = 4
SEQ_LEN = 8192
EMBD_DIM = 128
NUM_TOKENS = BATCH * SEQ_LEN

NUM_CORES = 2
NUM_SUBCORES = 16
NW = NUM_CORES * NUM_SUBCORES
TOK_PER_W = NUM_TOKENS // NW
CHUNK = 128
NCHUNK = TOK_PER_W // CHUNK
LANES = 16
NBUF = 5
DEPTH = 3
TROWS = SEQ_LEN // NW


def _sc_body(x_hbm, embd_hbm, pos_hbm, out_hbm, idx_v, pos_v, *scratch):
    bufs = scratch[:NBUF]
    osems = scratch[NBUF:2 * NBUF]
    gsems = scratch[2 * NBUF:2 * NBUF + DEPTH + 1]
    psem = scratch[-1]

    wid = lax.axis_index("s") * NUM_CORES + lax.axis_index("c")
    t0 = wid * TROWS

    pltpu.sync_copy(x_hbm.at[wid], idx_v)
    pos_cp = pltpu.async_copy(pos_hbm.at[pl.ds(t0, TROWS)], pos_v, psem)

    def gather_start(c):
        b = c % NBUF
        return pltpu.async_copy(
            embd_hbm.at[idx_v.at[c]], bufs[b], gsems[c % (DEPTH + 1)])

    def store_start(c):
        b = c % NBUF
        base = (c // 2) * SEQ_LEN + t0 + (c % 2) * CHUNK
        return pltpu.async_copy(
            bufs[b], out_hbm.at[pl.ds(base, CHUNK)], osems[b])

    def add_pos(c):
        buf = bufs[c % NBUF]
        po = (c % 2) * CHUNK

        @plsc.parallel_loop(0, CHUNK, unroll=8)
        def _(r):
            for j in range(EMBD_DIM // LANES):
                s = pl.ds(j * LANES, LANES)
                plsc.addupdate(buf.at[r, s], pos_v[po + r, s])

    g_cp = [None] * NCHUNK
    s_cp = [None] * NCHUNK
    s_waited = [False] * NCHUNK

    for c in range(min(DEPTH, NCHUNK)):
        g_cp[c] = gather_start(c)
    pos_cp.wait()
    for c in range(NCHUNK):
        nxt = c + DEPTH
        if nxt < NCHUNK:
            prev = nxt - NBUF
            if prev >= 0 and not s_waited[prev]:
                s_cp[prev].wait()
                s_waited[prev] = True
            g_cp[nxt] = gather_start(nxt)
        g_cp[c].wait()
        add_pos(c)
        s_cp[c] = store_start(c)
    for c in range(NCHUNK):
        if not s_waited[c]:
            s_cp[c].wait()
            s_waited[c] = True


@jax.jit
def kernel(x, embd_table, pos_table):
    xr = (x.astype(jnp.int32)
          .reshape(BATCH, NW, 2, CHUNK)
          .transpose(1, 0, 2, 3)
          .reshape(NW, NCHUNK, CHUNK))
    mesh = plsc.VectorSubcoreMesh(core_axis_name="c", subcore_axis_name="s")
    out = pl.kernel(
        _sc_body,
        out_type=jax.ShapeDtypeStruct((NUM_TOKENS, EMBD_DIM), jnp.float32),
        mesh=mesh,
        scratch_types=(
            [pltpu.VMEM((NCHUNK, CHUNK), jnp.int32),
             pltpu.VMEM((TROWS, EMBD_DIM), jnp.float32)]
            + [pltpu.VMEM((CHUNK, EMBD_DIM), jnp.float32)] * NBUF
            + [pltpu.SemaphoreType.DMA] * (NBUF + DEPTH + 2)
        ),
    )(xr, embd_table, pos_table)
    return out.reshape(BATCH, SEQ_LEN, EMBD_DIM)


# speedup vs baseline: 1.0649x; 1.0649x over previous
import functools

import jax
import jax.numpy as jnp
from jax import lax
from jax.experimental import pallas as pl
from jax.experimental.pallas import tpu as pltpu
from jax.experimental.pallas import tpu_sc as plsc

BATCH = 4
SEQ_LEN = 8192
EMBD_DIM = 128
NUM_TOKENS = BATCH * SEQ_LEN

NUM_CORES = 2
NUM_SUBCORES = 16
NW = NUM_CORES * NUM_SUBCORES
TOK_PER_W = NUM_TOKENS // NW
CHUNK = 128
NCHUNK = TOK_PER_W // CHUNK
LANES = 16
NBUF = 5
DEPTH = 3
TROWS = SEQ_LEN // NW


def _sc_body(x_hbm, embd_hbm, pos_hbm, out_hbm, idx_v, pos_v, *scratch):
    bufs = scratch[:NBUF]
    osems = scratch[NBUF:2 * NBUF]
    gsems = scratch[2 * NBUF:2 * NBUF + DEPTH + 1]
    psem = scratch[-1]

    wid = lax.axis_index("s") * NUM_CORES + lax.axis_index("c")
    t0 = wid * TROWS

    pltpu.sync_copy(x_hbm.at[wid], idx_v)
    pos_cp = pltpu.async_copy(pos_hbm.at[pl.ds(t0, TROWS)], pos_v, psem)

    def gather_start(c):
        b = c % NBUF
        return pltpu.async_copy(
            embd_hbm.at[idx_v.at[c]], bufs[b], gsems[c % (DEPTH + 1)])

    def store_start(c):
        b = c % NBUF
        base = (c // 2) * SEQ_LEN + t0 + (c % 2) * CHUNK
        return pltpu.async_copy(
            bufs[b], out_hbm.at[pl.ds(base, CHUNK)], osems[b])

    def add_pos(c):
        buf = bufs[c % NBUF]
        po = (c % 2) * CHUNK

        @plsc.parallel_loop(0, CHUNK, unroll=4)
        def _(r):
            for j in range(EMBD_DIM // LANES):
                s = pl.ds(j * LANES, LANES)
                plsc.addupdate(buf.at[r, s], pos_v[po + r, s])

    g_cp = [None] * NCHUNK
    s_cp = [None] * NCHUNK
    s_waited = [False] * NCHUNK

    for c in range(min(DEPTH, NCHUNK)):
        g_cp[c] = gather_start(c)
    pos_cp.wait()
    for c in range(NCHUNK):
        nxt = c + DEPTH
        if nxt < NCHUNK:
            prev = nxt - NBUF
            if prev >= 0 and not s_waited[prev]:
                s_cp[prev].wait()
                s_waited[prev] = True
            g_cp[nxt] = gather_start(nxt)
        g_cp[c].wait()
        add_pos(c)
        s_cp[c] = store_start(c)
    for c in range(NCHUNK):
        if not s_waited[c]:
            s_cp[c].wait()
            s_waited[c] = True


@jax.jit
def kernel(x, embd_table, pos_table):
    xr = (x.astype(jnp.int32)
          .reshape(BATCH, NW, 2, CHUNK)
          .transpose(1, 0, 2, 3)
          .reshape(NW, NCHUNK, CHUNK))
    mesh = plsc.VectorSubcoreMesh(core_axis_name="c", subcore_axis_name="s")
    out = pl.kernel(
        _sc_body,
        out_type=jax.ShapeDtypeStruct((NUM_TOKENS, EMBD_DIM), jnp.float32),
        mesh=mesh,
        scratch_types=(
            [pltpu.VMEM((NCHUNK, CHUNK), jnp.int32),
             pltpu.VMEM((TROWS, EMBD_DIM), jnp.float32)]
            + [pltpu.VMEM((CHUNK, EMBD_DIM), jnp.float32)] * NBUF
            + [pltpu.SemaphoreType.DMA] * (NBUF + DEPTH + 2)
        ),
    )(xr, embd_table, pos_table)
    return out.reshape(BATCH, SEQ_LEN, EMBD_DIM)
